# baseline (speedup 1.0000x reference)
import functools

import jax
import jax.numpy as jnp
from jax import lax
from jax.experimental import pallas as pl
from jax.experimental.pallas import tpu as pltpu
from jax.experimental.pallas import tpu_sc as plsc

D_MODEL = 64
SCALE = 8.0
VOCAB = 1000000
NUM_WORKERS = 32
BATCH = 4096
SEQ = 200
BLK = BATCH // NUM_WORKERS
LANES = 16
GBUF = 4
SBUF = 2
DT = D_MODEL // 8
ST = SEQ // 8
PAD = BLK // 2 + 1
CB = 1280
TGRID = -(-VOCAB // CB)


def _untile_body(tt_ref, out_ref):
    ident = (jnp.eye(D_MODEL, dtype=jnp.float32) * SCALE)
    x = tt_ref[...]
    y = lax.dot_general(
        x, ident, (((0,), (0,)), ((), ())),
        preferred_element_type=jnp.float32,
    )
    out_ref[...] = jnp.concatenate(
        [y, jnp.zeros((CB, 128 - D_MODEL), jnp.float32)], axis=1
    )


@jax.jit
def _untile(tt):
    return pl.pallas_call(
        _untile_body,
        out_shape=jax.ShapeDtypeStruct((VOCAB, 128), jnp.float32),
        grid=(TGRID,),
        in_specs=[pl.BlockSpec((D_MODEL, CB), lambda g: (0, g))],
        out_specs=pl.BlockSpec((CB, 128), lambda g: (g, 0)),
    )(tt)


def _emb_body(x4_hbm, table_hbm, out_hbm, idxt_v, rows_v, rpad_v, tbuf_v,
              gsem, ssem):
    c = lax.axis_index("c")
    s_ax = lax.axis_index("s")
    wid = s_ax * 2 + c

    pltpu.sync_copy(x4_hbm.at[:, wid], idxt_v)

    lane = lax.broadcasted_iota(jnp.int32, (LANES,), 0)
    rowidx = [lane + (j * LANES) for j in range(BLK // LANES)]

    def fire_gather(s, b):
        pltpu.async_copy(
            table_hbm.at[idxt_v.at[s // 8, s % 8]],
            rows_v.at[b],
            gsem.at[b],
        )

    def drain_gather(s, b):
        pltpu.make_async_copy(
            table_hbm.at[idxt_v.at[s // 8, s % 8]],
            rows_v.at[b],
            gsem.at[b],
        ).wait()

    for p in range(GBUF - 1):
        fire_gather(p, p)

    @pl.loop(0, SEQ, step=GBUF)
    def _pos_quad(t):
        for rb in range(GBUF):
            s = t + rb
            sb = rb % SBUF

            @pl.when(s + (GBUF - 1) < SEQ)
            def _():
                fire_gather(s + (GBUF - 1), (rb + GBUF - 1) % GBUF)

            drain_gather(s, rb)

            @plsc.parallel_loop(0, BLK, unroll=4)
            def _pad(i):
                for k in range(D_MODEL // LANES):
                    rpad_v[i, pl.ds(k * LANES, LANES)] = rows_v[
                        rb, i, pl.ds(k * LANES, LANES)
                    ]

            @pl.when(s >= SBUF)
            def _():
                pltpu.make_async_copy(
                    tbuf_v.at[sb], out_hbm.at[0, :, wid], ssem.at[sb]
                ).wait()

            @plsc.parallel_loop(0, D_MODEL, unroll=4)
            def _transpose(d):
                dt = d // 8
                di = d % 8
                dcol = jnp.full((LANES,), 0, jnp.int32) + d
                for j in range(BLK // LANES):
                    g = plsc.load_gather(rpad_v, [rowidx[j], dcol])
                    tbuf_v[sb, dt, di, pl.ds(j * LANES, LANES)] = g

            pltpu.async_copy(
                tbuf_v.at[sb], out_hbm.at[s, :, wid], ssem.at[sb]
            )

    for sb in range(SBUF):
        pltpu.make_async_copy(
            tbuf_v.at[sb], out_hbm.at[0, :, wid], ssem.at[sb]
        ).wait()


@jax.jit
def _emb(x4, table_pad):
    mesh = plsc.VectorSubcoreMesh(core_axis_name="c", subcore_axis_name="s")
    f = pl.kernel(
        _emb_body,
        mesh=mesh,
        out_type=jax.ShapeDtypeStruct(
            (SEQ, DT, NUM_WORKERS, 8, BLK), jnp.float32
        ),
        scratch_types=[
            pltpu.VMEM((ST, 8, BLK), jnp.int32),
            pltpu.VMEM((GBUF, BLK, 128), jnp.float32),
            pltpu.VMEM((BLK, PAD), jnp.float32),
            pltpu.VMEM((SBUF, DT, 8, BLK), jnp.float32),
            pltpu.SemaphoreType.DMA((GBUF,)),
            pltpu.SemaphoreType.DMA((SBUF,)),
        ],
        compiler_params=pltpu.CompilerParams(
            use_tc_tiling_on_sc=False, needs_layout_passes=False
        ),
    )
    return f(x4, table_pad)


def kernel(x, table):
    x4 = jnp.transpose(
        x.astype(jnp.int32).reshape(NUM_WORKERS, BLK, ST, 8), (2, 0, 3, 1)
    )
    table_pad = _untile(jnp.transpose(table))
    p5 = _emb(x4, table_pad)
    out = jnp.transpose(p5, (2, 4, 0, 1, 3))
    return out.reshape(BATCH, SEQ, D_MODEL)


# speedup vs baseline: 1.0342x; 1.0342x over previous
import functools

import jax
import jax.numpy as jnp
from jax import lax
from jax.experimental import pallas as pl
from jax.experimental.pallas import tpu as pltpu
from jax.experimental.pallas import tpu_sc as plsc

D_MODEL = 64
SCALE = 8.0
VOCAB = 1000000
NUM_WORKERS = 32
BATCH = 4096
SEQ = 200
BLK = BATCH // NUM_WORKERS
LANES = 16
GBUF = 4
SBUF = 2
DT = D_MODEL // 8
ST = SEQ // 8
PAD = BLK // 2 + 1
CB = 1280
TGRID = -(-VOCAB // CB)


def _untile_body(tt_ref, out_ref):
    x = tt_ref[...]
    y = jnp.transpose(x) * SCALE
    out_ref[...] = jnp.concatenate(
        [y, jnp.zeros((CB, 128 - D_MODEL), jnp.float32)], axis=1
    )


@jax.jit
def _untile(tt):
    return pl.pallas_call(
        _untile_body,
        out_shape=jax.ShapeDtypeStruct((VOCAB, 128), jnp.float32),
        grid=(TGRID,),
        in_specs=[pl.BlockSpec((D_MODEL, CB), lambda g: (0, g))],
        out_specs=pl.BlockSpec((CB, 128), lambda g: (g, 0)),
    )(tt)


def _emb_body(x4_hbm, table_hbm, out_hbm, idxt_v, rows_v, rpad_v, tbuf_v,
              gsem, ssem):
    c = lax.axis_index("c")
    s_ax = lax.axis_index("s")
    wid = s_ax * 2 + c

    pltpu.sync_copy(x4_hbm.at[:, wid], idxt_v)

    lane = lax.broadcasted_iota(jnp.int32, (LANES,), 0)
    rowidx = [lane + (j * LANES) for j in range(BLK // LANES)]

    def fire_gather(s, b):
        pltpu.async_copy(
            table_hbm.at[idxt_v.at[s // 8, s % 8]],
            rows_v.at[b],
            gsem.at[b],
        )

    def drain_gather(s, b):
        pltpu.make_async_copy(
            table_hbm.at[idxt_v.at[s // 8, s % 8]],
            rows_v.at[b],
            gsem.at[b],
        ).wait()

    for p in range(GBUF - 1):
        fire_gather(p, p)

    @pl.loop(0, SEQ, step=GBUF)
    def _pos_quad(t):
        for rb in range(GBUF):
            s = t + rb
            sb = rb % SBUF

            @pl.when(s + (GBUF - 1) < SEQ)
            def _():
                fire_gather(s + (GBUF - 1), (rb + GBUF - 1) % GBUF)

            drain_gather(s, rb)

            @plsc.parallel_loop(0, BLK, unroll=4)
            def _pad(i):
                for k in range(D_MODEL // LANES):
                    rpad_v[i, pl.ds(k * LANES, LANES)] = rows_v[
                        rb, i, pl.ds(k * LANES, LANES)
                    ]

            @pl.when(s >= SBUF)
            def _():
                pltpu.make_async_copy(
                    tbuf_v.at[sb], out_hbm.at[0, :, wid], ssem.at[sb]
                ).wait()

            @plsc.parallel_loop(0, D_MODEL, unroll=4)
            def _transpose(d):
                dt = d // 8
                di = d % 8
                dcol = jnp.full((LANES,), 0, jnp.int32) + d
                for j in range(BLK // LANES):
                    g = plsc.load_gather(rpad_v, [rowidx[j], dcol])
                    tbuf_v[sb, dt, di, pl.ds(j * LANES, LANES)] = g

            pltpu.async_copy(
                tbuf_v.at[sb], out_hbm.at[s, :, wid], ssem.at[sb]
            )

    for sb in range(SBUF):
        pltpu.make_async_copy(
            tbuf_v.at[sb], out_hbm.at[0, :, wid], ssem.at[sb]
        ).wait()


@jax.jit
def _emb(x4, table_pad):
    mesh = plsc.VectorSubcoreMesh(core_axis_name="c", subcore_axis_name="s")
    f = pl.kernel(
        _emb_body,
        mesh=mesh,
        out_type=jax.ShapeDtypeStruct(
            (SEQ, DT, NUM_WORKERS, 8, BLK), jnp.float32
        ),
        scratch_types=[
            pltpu.VMEM((ST, 8, BLK), jnp.int32),
            pltpu.VMEM((GBUF, BLK, 128), jnp.float32),
            pltpu.VMEM((BLK, PAD), jnp.float32),
            pltpu.VMEM((SBUF, DT, 8, BLK), jnp.float32),
            pltpu.SemaphoreType.DMA((GBUF,)),
            pltpu.SemaphoreType.DMA((SBUF,)),
        ],
        compiler_params=pltpu.CompilerParams(
            use_tc_tiling_on_sc=False, needs_layout_passes=False
        ),
    )
    return f(x4, table_pad)


def kernel(x, table):
    x4 = jnp.transpose(
        x.astype(jnp.int32).reshape(NUM_WORKERS, BLK, ST, 8), (2, 0, 3, 1)
    )
    table_pad = _untile(jnp.transpose(table))
    p5 = _emb(x4, table_pad)
    out = jnp.transpose(p5, (2, 4, 0, 1, 3))
    return out.reshape(BATCH, SEQ, D_MODEL)


# speedup vs baseline: 1.5405x; 1.4895x over previous
import functools

import jax
import jax.numpy as jnp
from jax import lax
from jax.experimental import pallas as pl
from jax.experimental.pallas import tpu as pltpu
from jax.experimental.pallas import tpu_sc as plsc

D_MODEL = 64
SCALE = 8.0
VOCAB = 1000000
NUM_WORKERS = 32
BATCH = 4096
SEQ = 200
BLK = BATCH // NUM_WORKERS
LANES = 16
GBUF = 4
SBUF = 2
DT = D_MODEL // 8
ST = SEQ // 8
PAD = BLK // 2 + 1
CB = 4096
TGRID = -(-VOCAB // CB)


def _untile_body(tt_ref, out_ref):
    x = tt_ref[...]
    y = jnp.transpose(x) * SCALE
    out_ref[...] = jnp.concatenate(
        [y, jnp.zeros((CB, 128 - D_MODEL), jnp.float32)], axis=1
    )


@jax.jit
def _untile(tt):
    return pl.pallas_call(
        _untile_body,
        out_shape=jax.ShapeDtypeStruct((VOCAB, 128), jnp.float32),
        grid=(TGRID,),
        in_specs=[pl.BlockSpec((D_MODEL, CB), lambda g: (0, g))],
        out_specs=pl.BlockSpec((CB, 128), lambda g: (g, 0)),
    )(tt)


def _emb_body(x4_hbm, table_hbm, out_hbm, idxt_v, rows_v, rpad_v, tbuf_v,
              gsem, ssem):
    c = lax.axis_index("c")
    s_ax = lax.axis_index("s")
    wid = s_ax * 2 + c

    pltpu.sync_copy(x4_hbm.at[:, wid], idxt_v)

    lane = lax.broadcasted_iota(jnp.int32, (LANES,), 0)
    rowidx = [lane + (j * LANES) for j in range(BLK // LANES)]

    def fire_gather(s, b):
        pltpu.async_copy(
            table_hbm.at[idxt_v.at[s // 8, s % 8]],
            rows_v.at[b],
            gsem.at[b],
        )

    def drain_gather(s, b):
        pltpu.make_async_copy(
            table_hbm.at[idxt_v.at[s // 8, s % 8]],
            rows_v.at[b],
            gsem.at[b],
        ).wait()

    for p in range(GBUF - 1):
        fire_gather(p, p)

    @pl.loop(0, SEQ, step=GBUF)
    def _pos_quad(t):
        for rb in range(GBUF):
            s = t + rb
            sb = rb % SBUF

            @pl.when(s + (GBUF - 1) < SEQ)
            def _():
                fire_gather(s + (GBUF - 1), (rb + GBUF - 1) % GBUF)

            drain_gather(s, rb)

            @plsc.parallel_loop(0, BLK, unroll=4)
            def _pad(i):
                for k in range(D_MODEL // LANES):
                    rpad_v[i, pl.ds(k * LANES, LANES)] = rows_v[
                        rb, i, pl.ds(k * LANES, LANES)
                    ]

            @pl.when(s >= SBUF)
            def _():
                pltpu.make_async_copy(
                    tbuf_v.at[sb], out_hbm.at[0, :, wid], ssem.at[sb]
                ).wait()

            @plsc.parallel_loop(0, D_MODEL, unroll=4)
            def _transpose(d):
                dt = d // 8
                di = d % 8
                dcol = jnp.full((LANES,), 0, jnp.int32) + d
                for j in range(BLK // LANES):
                    g = plsc.load_gather(rpad_v, [rowidx[j], dcol])
                    tbuf_v[sb, dt, di, pl.ds(j * LANES, LANES)] = g

            pltpu.async_copy(
                tbuf_v.at[sb], out_hbm.at[s, :, wid], ssem.at[sb]
            )

    for sb in range(SBUF):
        pltpu.make_async_copy(
            tbuf_v.at[sb], out_hbm.at[0, :, wid], ssem.at[sb]
        ).wait()


@jax.jit
def _emb(x4, table_pad):
    mesh = plsc.VectorSubcoreMesh(core_axis_name="c", subcore_axis_name="s")
    f = pl.kernel(
        _emb_body,
        mesh=mesh,
        out_type=jax.ShapeDtypeStruct(
            (SEQ, DT, NUM_WORKERS, 8, BLK), jnp.float32
        ),
        scratch_types=[
            pltpu.VMEM((ST, 8, BLK), jnp.int32),
            pltpu.VMEM((GBUF, BLK, 128), jnp.float32),
            pltpu.VMEM((BLK, PAD), jnp.float32),
            pltpu.VMEM((SBUF, DT, 8, BLK), jnp.float32),
            pltpu.SemaphoreType.DMA((GBUF,)),
            pltpu.SemaphoreType.DMA((SBUF,)),
        ],
        compiler_params=pltpu.CompilerParams(
            use_tc_tiling_on_sc=False, needs_layout_passes=False
        ),
    )
    return f(x4, table_pad)


def kernel(x, table):
    x4 = jnp.transpose(
        x.astype(jnp.int32).reshape(NUM_WORKERS, BLK, ST, 8), (2, 0, 3, 1)
    )
    table_pad = _untile(jnp.transpose(table))
    p5 = _emb(x4, table_pad)
    out = jnp.transpose(p5, (2, 4, 0, 1, 3))
    return out.reshape(BATCH, SEQ, D_MODEL)


# speedup vs baseline: 1.7606x; 1.1429x over previous
import functools

import jax
import jax.numpy as jnp
from jax import lax
from jax.experimental import pallas as pl
from jax.experimental.pallas import tpu as pltpu
from jax.experimental.pallas import tpu_sc as plsc

D_MODEL = 64
SCALE = 8.0
VOCAB = 1000000
NUM_WORKERS = 32
BATCH = 4096
SEQ = 200
BLK = BATCH // NUM_WORKERS
LANES = 16
GBUF = 4
SBUF = 2
DT = D_MODEL // 8
ST = SEQ // 8
PAD = BLK // 2 + 1
CB = 8192
TGRID = -(-VOCAB // CB)


def _untile_body(tt_ref, out_ref):
    x = tt_ref[...]
    y = jnp.transpose(x) * SCALE
    out_ref[...] = jnp.concatenate(
        [y, jnp.zeros((CB, 128 - D_MODEL), jnp.float32)], axis=1
    )


@jax.jit
def _untile(tt):
    return pl.pallas_call(
        _untile_body,
        out_shape=jax.ShapeDtypeStruct((VOCAB, 128), jnp.float32),
        grid=(TGRID,),
        in_specs=[pl.BlockSpec((D_MODEL, CB), lambda g: (0, g))],
        out_specs=pl.BlockSpec((CB, 128), lambda g: (g, 0)),
    )(tt)


def _emb_body(x4_hbm, table_hbm, out_hbm, idxt_v, rows_v, rpad_v, tbuf_v,
              gsem, ssem):
    c = lax.axis_index("c")
    s_ax = lax.axis_index("s")
    wid = s_ax * 2 + c

    pltpu.sync_copy(x4_hbm.at[:, wid], idxt_v)

    lane = lax.broadcasted_iota(jnp.int32, (LANES,), 0)
    rowidx = [lane + (j * LANES) for j in range(BLK // LANES)]

    def fire_gather(s, b):
        pltpu.async_copy(
            table_hbm.at[idxt_v.at[s // 8, s % 8]],
            rows_v.at[b],
            gsem.at[b],
        )

    def drain_gather(s, b):
        pltpu.make_async_copy(
            table_hbm.at[idxt_v.at[s // 8, s % 8]],
            rows_v.at[b],
            gsem.at[b],
        ).wait()

    for p in range(GBUF - 1):
        fire_gather(p, p)

    @pl.loop(0, SEQ, step=GBUF)
    def _pos_quad(t):
        for rb in range(GBUF):
            s = t + rb
            sb = rb % SBUF

            @pl.when(s + (GBUF - 1) < SEQ)
            def _():
                fire_gather(s + (GBUF - 1), (rb + GBUF - 1) % GBUF)

            drain_gather(s, rb)

            @plsc.parallel_loop(0, BLK, unroll=4)
            def _pad(i):
                for k in range(D_MODEL // LANES):
                    rpad_v[i, pl.ds(k * LANES, LANES)] = rows_v[
                        rb, i, pl.ds(k * LANES, LANES)
                    ]

            @pl.when(s >= SBUF)
            def _():
                pltpu.make_async_copy(
                    tbuf_v.at[sb], out_hbm.at[0, :, wid], ssem.at[sb]
                ).wait()

            @plsc.parallel_loop(0, D_MODEL, unroll=4)
            def _transpose(d):
                dt = d // 8
                di = d % 8
                dcol = jnp.full((LANES,), 0, jnp.int32) + d
                for j in range(BLK // LANES):
                    g = plsc.load_gather(rpad_v, [rowidx[j], dcol])
                    tbuf_v[sb, dt, di, pl.ds(j * LANES, LANES)] = g

            pltpu.async_copy(
                tbuf_v.at[sb], out_hbm.at[s, :, wid], ssem.at[sb]
            )

    for sb in range(SBUF):
        pltpu.make_async_copy(
            tbuf_v.at[sb], out_hbm.at[0, :, wid], ssem.at[sb]
        ).wait()


@jax.jit
def _emb(x4, table_pad):
    mesh = plsc.VectorSubcoreMesh(core_axis_name="c", subcore_axis_name="s")
    f = pl.kernel(
        _emb_body,
        mesh=mesh,
        out_type=jax.ShapeDtypeStruct(
            (SEQ, DT, NUM_WORKERS, 8, BLK), jnp.float32
        ),
        scratch_types=[
            pltpu.VMEM((ST, 8, BLK), jnp.int32),
            pltpu.VMEM((GBUF, BLK, 128), jnp.float32),
            pltpu.VMEM((BLK, PAD), jnp.float32),
            pltpu.VMEM((SBUF, DT, 8, BLK), jnp.float32),
            pltpu.SemaphoreType.DMA((GBUF,)),
            pltpu.SemaphoreType.DMA((SBUF,)),
        ],
        compiler_params=pltpu.CompilerParams(
            use_tc_tiling_on_sc=False, needs_layout_passes=False
        ),
    )
    return f(x4, table_pad)


def kernel(x, table):
    x4 = jnp.transpose(
        x.astype(jnp.int32).reshape(NUM_WORKERS, BLK, ST, 8), (2, 0, 3, 1)
    )
    table_pad = _untile(jnp.transpose(table))
    p5 = _emb(x4, table_pad)
    out = jnp.transpose(p5, (2, 4, 0, 1, 3))
    return out.reshape(BATCH, SEQ, D_MODEL)


# speedup vs baseline: 1.8552x; 1.0537x over previous
import functools

import jax
import jax.numpy as jnp
from jax import lax
from jax.experimental import pallas as pl
from jax.experimental.pallas import tpu as pltpu
from jax.experimental.pallas import tpu_sc as plsc

D_MODEL = 64
SCALE = 8.0
VOCAB = 1000000
NUM_WORKERS = 32
BATCH = 4096
SEQ = 200
BLK = BATCH // NUM_WORKERS
LANES = 16
GBUF = 4
SBUF = 2
DT = D_MODEL // 8
ST = SEQ // 8
PAD = BLK // 2 + 1
CB = 25600
TGRID = -(-VOCAB // CB)


def _untile_body(tt_ref, out_ref):
    x = tt_ref[...]
    y = jnp.transpose(x) * SCALE
    out_ref[...] = jnp.concatenate(
        [y, jnp.zeros((CB, 128 - D_MODEL), jnp.float32)], axis=1
    )


@jax.jit
def _untile(tt):
    return pl.pallas_call(
        _untile_body,
        out_shape=jax.ShapeDtypeStruct((VOCAB, 128), jnp.float32),
        grid=(TGRID,),
        in_specs=[pl.BlockSpec((D_MODEL, CB), lambda g: (0, g))],
        out_specs=pl.BlockSpec((CB, 128), lambda g: (g, 0)),
    )(tt)


def _emb_body(x4_hbm, table_hbm, out_hbm, idxt_v, rows_v, rpad_v, tbuf_v,
              gsem, ssem):
    c = lax.axis_index("c")
    s_ax = lax.axis_index("s")
    wid = s_ax * 2 + c

    pltpu.sync_copy(x4_hbm.at[:, wid], idxt_v)

    lane = lax.broadcasted_iota(jnp.int32, (LANES,), 0)
    rowidx = [lane + (j * LANES) for j in range(BLK // LANES)]

    def fire_gather(s, b):
        pltpu.async_copy(
            table_hbm.at[idxt_v.at[s // 8, s % 8]],
            rows_v.at[b],
            gsem.at[b],
        )

    def drain_gather(s, b):
        pltpu.make_async_copy(
            table_hbm.at[idxt_v.at[s // 8, s % 8]],
            rows_v.at[b],
            gsem.at[b],
        ).wait()

    for p in range(GBUF - 1):
        fire_gather(p, p)

    @pl.loop(0, SEQ, step=GBUF)
    def _pos_quad(t):
        for rb in range(GBUF):
            s = t + rb
            sb = rb % SBUF

            @pl.when(s + (GBUF - 1) < SEQ)
            def _():
                fire_gather(s + (GBUF - 1), (rb + GBUF - 1) % GBUF)

            drain_gather(s, rb)

            @plsc.parallel_loop(0, BLK, unroll=4)
            def _pad(i):
                for k in range(D_MODEL // LANES):
                    rpad_v[i, pl.ds(k * LANES, LANES)] = rows_v[
                        rb, i, pl.ds(k * LANES, LANES)
                    ]

            @pl.when(s >= SBUF)
            def _():
                pltpu.make_async_copy(
                    tbuf_v.at[sb], out_hbm.at[0, :, wid], ssem.at[sb]
                ).wait()

            @plsc.parallel_loop(0, D_MODEL, unroll=4)
            def _transpose(d):
                dt = d // 8
                di = d % 8
                dcol = jnp.full((LANES,), 0, jnp.int32) + d
                for j in range(BLK // LANES):
                    g = plsc.load_gather(rpad_v, [rowidx[j], dcol])
                    tbuf_v[sb, dt, di, pl.ds(j * LANES, LANES)] = g

            pltpu.async_copy(
                tbuf_v.at[sb], out_hbm.at[s, :, wid], ssem.at[sb]
            )

    for sb in range(SBUF):
        pltpu.make_async_copy(
            tbuf_v.at[sb], out_hbm.at[0, :, wid], ssem.at[sb]
        ).wait()


@jax.jit
def _emb(x4, table_pad):
    mesh = plsc.VectorSubcoreMesh(core_axis_name="c", subcore_axis_name="s")
    f = pl.kernel(
        _emb_body,
        mesh=mesh,
        out_type=jax.ShapeDtypeStruct(
            (SEQ, DT, NUM_WORKERS, 8, BLK), jnp.float32
        ),
        scratch_types=[
            pltpu.VMEM((ST, 8, BLK), jnp.int32),
            pltpu.VMEM((GBUF, BLK, 128), jnp.float32),
            pltpu.VMEM((BLK, PAD), jnp.float32),
            pltpu.VMEM((SBUF, DT, 8, BLK), jnp.float32),
            pltpu.SemaphoreType.DMA((GBUF,)),
            pltpu.SemaphoreType.DMA((SBUF,)),
        ],
        compiler_params=pltpu.CompilerParams(
            use_tc_tiling_on_sc=False, needs_layout_passes=False
        ),
    )
    return f(x4, table_pad)


def kernel(x, table):
    x4 = jnp.transpose(
        x.astype(jnp.int32).reshape(NUM_WORKERS, BLK, ST, 8), (2, 0, 3, 1)
    )
    table_pad = _untile(jnp.transpose(table))
    p5 = _emb(x4, table_pad)
    out = jnp.transpose(p5, (2, 4, 0, 1, 3))
    return out.reshape(BATCH, SEQ, D_MODEL)
